# baseline (speedup 1.0000x reference)
import functools

import jax
import jax.numpy as jnp
from jax import lax
from jax.experimental import pallas as pl
from jax.experimental.pallas import tpu as pltpu
from jax.experimental.pallas import tpu_sc as plsc

N = 10000
E = 160000
DC = 128
NS = 16
NCORE = 2
CHUNK = 128
KT = 80
HKT = KT // 2
EPAD = NS * KT * CHUNK
NACC = 10112
ZROW = NACC // NS


def _spmm_body(nblk, with_deg, *refs):
    if with_deg:
        (x_hbm, src_hbm, dst_hbm, zeros_hbm, out_hbm, deg_hbm,
         src_v, dst_v, rows_v, acc) = refs
    else:
        (x_hbm, src_hbm, dst_hbm, zeros_hbm, out_hbm,
         src_v, dst_v, rows_v, acc) = refs

    c = lax.axis_index("c")
    s = lax.axis_index("s")
    r0 = pl.multiple_of(s * ZROW, 8)

    pltpu.sync_copy(src_hbm.at[s], src_v)
    pltpu.sync_copy(dst_hbm.at[s], dst_v)

    bps = nblk // NCORE

    def _xrow(j, carry):
        for u in range(CHUNK // 16):
            sl = pl.ds(u * 16, 16)
            src_v[j, sl] = src_v[j, sl] * nblk + c * bps
        return carry
    lax.fori_loop(0, KT, _xrow, 0)

    for bb in range(bps):
        blk = c * bps + bb
        c0 = pl.multiple_of(blk * DC, DC)

        if bb > 0:
            def _bump(j, carry):
                for u in range(CHUNK // 16):
                    sl = pl.ds(u * 16, 16)
                    src_v[j, sl] = src_v[j, sl] + 1
                return carry
            lax.fori_loop(0, KT, _bump, 0)

        pltpu.sync_copy(zeros_hbm.at[pl.ds(r0, ZROW), :],
                        acc.at[pl.ds(r0, ZROW), :])
        plsc.subcore_barrier()

        def _chunk(j, carry):
            pltpu.sync_copy(x_hbm.at[src_v.at[j]], rows_v)
            pltpu.sync_copy(rows_v, acc.at[dst_v.at[j]], add=True)
            return carry
        lax.fori_loop(0, KT, _chunk, 0)
        plsc.subcore_barrier()

        pltpu.sync_copy(acc.at[pl.ds(r0, ZROW), :],
                        out_hbm.at[pl.ds(r0, ZROW), pl.ds(c0, DC)])
        plsc.subcore_barrier()

    if with_deg:
        pltpu.sync_copy(zeros_hbm.at[pl.ds(r0, ZROW), :],
                        acc.at[pl.ds(r0, ZROW), :])

        def _fill_ones(i, carry):
            for u in range(DC // 16):
                rows_v[i, pl.ds(u * 16, 16)] = jnp.ones((16,), jnp.float32)
            return carry
        lax.fori_loop(0, CHUNK, _fill_ones, 0)
        plsc.subcore_barrier()

        base = c * HKT

        def _deg_chunk(j, carry):
            pltpu.sync_copy(rows_v, acc.at[dst_v.at[base + j]], add=True)
            return carry
        lax.fori_loop(0, HKT, _deg_chunk, 0)
        plsc.subcore_barrier()

        cdeg = pl.multiple_of(c * DC, DC)
        pltpu.sync_copy(acc.at[pl.ds(r0, ZROW), :],
                        deg_hbm.at[pl.ds(r0, ZROW), pl.ds(cdeg, DC)])
        plsc.subcore_barrier()


def _make_spmm(nblk, with_deg):
    d = nblk * DC
    out_type = [jax.ShapeDtypeStruct((NACC, d), jnp.float32)]
    if with_deg:
        out_type.append(jax.ShapeDtypeStruct((NACC, 2 * DC), jnp.float32))
    scratch = [
        pltpu.VMEM((KT, CHUNK), jnp.int32),
        pltpu.VMEM((KT, CHUNK), jnp.int32),
        pltpu.VMEM((CHUNK, DC), jnp.float32),
        pltpu.VMEM_SHARED((NACC, DC), jnp.float32),
    ]
    mesh = plsc.VectorSubcoreMesh(core_axis_name="c", subcore_axis_name="s")
    return pl.kernel(
        functools.partial(_spmm_body, nblk, with_deg),
        out_type=tuple(out_type),
        mesh=mesh,
        scratch_types=scratch,
    )


RT = 400


def _deg_col(deg_ref):
    return jnp.maximum(deg_ref[:, 0:1] + deg_ref[:, DC:DC + 1], 1.0)


def _tc0_body(agg_ref, deg_ref, w0_ref, b0_ref, w1_ref, h0_ref, z1_ref):
    x = agg_ref[...] / _deg_col(deg_ref)
    h0 = jnp.dot(x, w0_ref[...], preferred_element_type=jnp.float32)
    h0 = jnp.maximum(h0 + b0_ref[...], 0.0)
    h0_ref[...] = h0
    z1_ref[...] = jnp.dot(h0, w1_ref[...], preferred_element_type=jnp.float32)


def _tc1_body(agg_ref, deg_ref, b1_ref, h0_ref, wo_ref, z2_ref):
    h = jnp.maximum(agg_ref[...] / _deg_col(deg_ref) + b1_ref[...], 0.0)
    h = h + h0_ref[...]
    z2_ref[...] = jnp.dot(h, wo_ref[...], preferred_element_type=jnp.float32)


def _tc2_body(agg_ref, deg_ref, bo_ref, out_ref):
    out_ref[...] = agg_ref[...] / _deg_col(deg_ref) + bo_ref[...]


def _row_spec(cols):
    return pl.BlockSpec((RT, cols), lambda r: (r, 0))


def _full_spec(rows, cols):
    return pl.BlockSpec((rows, cols), lambda r: (0, 0))


def _tc0(agg0, deg, W0, b0, W1):
    return pl.pallas_call(
        _tc0_body,
        grid=(N // RT,),
        in_specs=[_row_spec(256), _row_spec(2 * DC), _full_spec(256, 512),
                  _full_spec(1, 512), _full_spec(512, 512)],
        out_specs=[_row_spec(512), _row_spec(512)],
        out_shape=[jax.ShapeDtypeStruct((N, 512), jnp.float32),
                   jax.ShapeDtypeStruct((N, 512), jnp.float32)],
    )(agg0, deg, W0, b0, W1)


def _tc1(agg1, deg, b1, h0, Wo):
    return pl.pallas_call(
        _tc1_body,
        grid=(N // RT,),
        in_specs=[_row_spec(512), _row_spec(2 * DC), _full_spec(1, 512),
                  _row_spec(512), _full_spec(512, 256)],
        out_specs=_row_spec(256),
        out_shape=jax.ShapeDtypeStruct((N, 256), jnp.float32),
    )(agg1, deg, b1, h0, Wo)


def _tc2(agg2, deg, bo):
    return pl.pallas_call(
        _tc2_body,
        grid=(N // RT,),
        in_specs=[_row_spec(256), _row_spec(2 * DC), _full_spec(1, 256)],
        out_specs=_row_spec(256),
        out_shape=jax.ShapeDtypeStruct((N, 256), jnp.float32),
    )(agg2, deg, bo)


def kernel(feats, edge_index, W0, b0, W1, b1, Wo, bo):
    src = edge_index[0]
    dst = edge_index[1]
    pad = EPAD - E
    src2 = jnp.concatenate([src, jnp.zeros((pad,), jnp.int32)]).reshape(NS, KT, CHUNK)
    dst2 = jnp.concatenate([dst, jnp.full((pad,), N, jnp.int32)]).reshape(NS, KT, CHUNK)
    zeros = jnp.zeros((NACC, DC), jnp.float32)

    agg0, deg = _make_spmm(2, True)(
        feats.reshape(N * 2, DC), src2, dst2, zeros)
    h0, z1 = _tc0(agg0, deg, W0, b0.reshape(1, -1), W1)
    (agg1,) = _make_spmm(4, False)(z1.reshape(N * 4, DC), src2, dst2, zeros)
    z2 = _tc1(agg1, deg, b1.reshape(1, -1), h0, Wo)
    (agg2,) = _make_spmm(2, False)(z2.reshape(N * 2, DC), src2, dst2, zeros)
    return _tc2(agg2, deg, bo.reshape(1, -1))


# speedup vs baseline: 1.0098x; 1.0098x over previous
import functools

import jax
import jax.numpy as jnp
from jax import lax
from jax.experimental import pallas as pl
from jax.experimental.pallas import tpu as pltpu
from jax.experimental.pallas import tpu_sc as plsc

N = 10000
E = 160000
DC = 128
NS = 16
NCORE = 2
CHUNK = 128
KT = 80
HKT = KT // 2
EPAD = NS * KT * CHUNK
NACC = 10112
ZROW = NACC // NS


def _spmm_body(nblk, with_deg, *refs):
    if with_deg:
        (x_hbm, src_hbm, dst_hbm, zeros_hbm, out_hbm, deg_hbm,
         src_v, dst_v, rows_v, acc, gsem) = refs
    else:
        (x_hbm, src_hbm, dst_hbm, zeros_hbm, out_hbm,
         src_v, dst_v, rows_v, acc, gsem) = refs

    c = lax.axis_index("c")
    s = lax.axis_index("s")
    r0 = pl.multiple_of(s * ZROW, 8)

    pltpu.sync_copy(src_hbm.at[s], src_v)
    pltpu.sync_copy(dst_hbm.at[s], dst_v)

    bps = nblk // NCORE

    def _xrow(j, carry):
        for u in range(CHUNK // 16):
            sl = pl.ds(u * 16, 16)
            src_v[j, sl] = src_v[j, sl] * nblk + c * bps
        return carry
    lax.fori_loop(0, KT, _xrow, 0)

    for bb in range(bps):
        blk = c * bps + bb
        c0 = pl.multiple_of(blk * DC, DC)

        if bb > 0:
            def _bump(j, carry):
                for u in range(CHUNK // 16):
                    sl = pl.ds(u * 16, 16)
                    src_v[j, sl] = src_v[j, sl] + 1
                return carry
            lax.fori_loop(0, KT, _bump, 0)

        pltpu.sync_copy(zeros_hbm.at[pl.ds(r0, ZROW), :],
                        acc.at[pl.ds(r0, ZROW), :])
        plsc.subcore_barrier()

        def _chunk(j, carry):
            pltpu.async_copy(x_hbm.at[src_v.at[j]], rows_v, gsem).wait()
            pltpu.sync_copy(rows_v, acc.at[dst_v.at[j]], add=True)
            return carry
        lax.fori_loop(0, KT, _chunk, 0)
        plsc.subcore_barrier()

        pltpu.sync_copy(acc.at[pl.ds(r0, ZROW), :],
                        out_hbm.at[pl.ds(r0, ZROW), pl.ds(c0, DC)])
        plsc.subcore_barrier()

    if with_deg:
        pltpu.sync_copy(zeros_hbm.at[pl.ds(r0, ZROW), :],
                        acc.at[pl.ds(r0, ZROW), :])

        def _fill_ones(i, carry):
            for u in range(DC // 16):
                rows_v[i, pl.ds(u * 16, 16)] = jnp.ones((16,), jnp.float32)
            return carry
        lax.fori_loop(0, CHUNK, _fill_ones, 0)
        plsc.subcore_barrier()

        base = c * HKT

        def _deg_chunk(j, carry):
            pltpu.sync_copy(rows_v, acc.at[dst_v.at[base + j]], add=True)
            return carry
        lax.fori_loop(0, HKT, _deg_chunk, 0)
        plsc.subcore_barrier()

        cdeg = pl.multiple_of(c * DC, DC)
        pltpu.sync_copy(acc.at[pl.ds(r0, ZROW), :],
                        deg_hbm.at[pl.ds(r0, ZROW), pl.ds(cdeg, DC)])
        plsc.subcore_barrier()


def _make_spmm(nblk, with_deg):
    d = nblk * DC
    out_type = [jax.ShapeDtypeStruct((NACC, d), jnp.float32)]
    if with_deg:
        out_type.append(jax.ShapeDtypeStruct((NACC, 2 * DC), jnp.float32))
    scratch = [
        pltpu.VMEM((KT, CHUNK), jnp.int32),
        pltpu.VMEM((KT, CHUNK), jnp.int32),
        pltpu.VMEM((CHUNK, DC), jnp.float32),
        pltpu.VMEM_SHARED((NACC, DC), jnp.float32),
        pltpu.SemaphoreType.DMA,
    ]
    mesh = plsc.VectorSubcoreMesh(core_axis_name="c", subcore_axis_name="s")
    return pl.kernel(
        functools.partial(_spmm_body, nblk, with_deg),
        out_type=tuple(out_type),
        mesh=mesh,
        scratch_types=scratch,
    )


RT = 400


def _deg_col(deg_ref):
    return jnp.maximum(deg_ref[:, 0:1] + deg_ref[:, DC:DC + 1], 1.0)


def _tc0_body(agg_ref, deg_ref, w0_ref, b0_ref, w1_ref, h0_ref, z1_ref):
    x = agg_ref[...] / _deg_col(deg_ref)
    h0 = jnp.dot(x, w0_ref[...], preferred_element_type=jnp.float32)
    h0 = jnp.maximum(h0 + b0_ref[...], 0.0)
    h0_ref[...] = h0
    z1_ref[...] = jnp.dot(h0, w1_ref[...], preferred_element_type=jnp.float32)


def _tc1_body(agg_ref, deg_ref, b1_ref, h0_ref, wo_ref, z2_ref):
    h = jnp.maximum(agg_ref[...] / _deg_col(deg_ref) + b1_ref[...], 0.0)
    h = h + h0_ref[...]
    z2_ref[...] = jnp.dot(h, wo_ref[...], preferred_element_type=jnp.float32)


def _tc2_body(agg_ref, deg_ref, bo_ref, out_ref):
    out_ref[...] = agg_ref[...] / _deg_col(deg_ref) + bo_ref[...]


def _row_spec(cols):
    return pl.BlockSpec((RT, cols), lambda r: (r, 0))


def _full_spec(rows, cols):
    return pl.BlockSpec((rows, cols), lambda r: (0, 0))


def _tc0(agg0, deg, W0, b0, W1):
    return pl.pallas_call(
        _tc0_body,
        grid=(N // RT,),
        in_specs=[_row_spec(256), _row_spec(2 * DC), _full_spec(256, 512),
                  _full_spec(1, 512), _full_spec(512, 512)],
        out_specs=[_row_spec(512), _row_spec(512)],
        out_shape=[jax.ShapeDtypeStruct((N, 512), jnp.float32),
                   jax.ShapeDtypeStruct((N, 512), jnp.float32)],
    )(agg0, deg, W0, b0, W1)


def _tc1(agg1, deg, b1, h0, Wo):
    return pl.pallas_call(
        _tc1_body,
        grid=(N // RT,),
        in_specs=[_row_spec(512), _row_spec(2 * DC), _full_spec(1, 512),
                  _row_spec(512), _full_spec(512, 256)],
        out_specs=_row_spec(256),
        out_shape=jax.ShapeDtypeStruct((N, 256), jnp.float32),
    )(agg1, deg, b1, h0, Wo)


def _tc2(agg2, deg, bo):
    return pl.pallas_call(
        _tc2_body,
        grid=(N // RT,),
        in_specs=[_row_spec(256), _row_spec(2 * DC), _full_spec(1, 256)],
        out_specs=_row_spec(256),
        out_shape=jax.ShapeDtypeStruct((N, 256), jnp.float32),
    )(agg2, deg, bo)


def kernel(feats, edge_index, W0, b0, W1, b1, Wo, bo):
    src = edge_index[0]
    dst = edge_index[1]
    pad = EPAD - E
    src2 = jnp.concatenate([src, jnp.zeros((pad,), jnp.int32)]).reshape(NS, KT, CHUNK)
    dst2 = jnp.concatenate([dst, jnp.full((pad,), N, jnp.int32)]).reshape(NS, KT, CHUNK)
    zeros = jnp.zeros((NACC, DC), jnp.float32)

    agg0, deg = _make_spmm(2, True)(
        feats.reshape(N * 2, DC), src2, dst2, zeros)
    h0, z1 = _tc0(agg0, deg, W0, b0.reshape(1, -1), W1)
    (agg1,) = _make_spmm(4, False)(z1.reshape(N * 4, DC), src2, dst2, zeros)
    z2 = _tc1(agg1, deg, b1.reshape(1, -1), h0, Wo)
    (agg2,) = _make_spmm(2, False)(z2.reshape(N * 2, DC), src2, dst2, zeros)
    return _tc2(agg2, deg, bo.reshape(1, -1))


# speedup vs baseline: 1.8112x; 1.7936x over previous
import functools

import jax
import jax.numpy as jnp
from jax import lax
from jax.experimental import pallas as pl
from jax.experimental.pallas import tpu as pltpu
from jax.experimental.pallas import tpu_sc as plsc

N = 10000
E = 160000
DC = 128
NS = 16
NCORE = 2
CHUNK = 128
KT = 80
HKT = KT // 2
EPAD = NS * KT * CHUNK
NACC = 10112
ZROW = NACC // NS


def _spmm_body(nblk, with_deg, *refs):
    if with_deg:
        (x_hbm, src_hbm, dst_hbm, zeros_hbm, out_hbm, deg_hbm,
         src_v, dst_v, rows_v, acc, gsem) = refs
    else:
        (x_hbm, src_hbm, dst_hbm, zeros_hbm, out_hbm,
         src_v, dst_v, rows_v, acc, gsem) = refs

    c = lax.axis_index("c")
    s = lax.axis_index("s")
    r0 = pl.multiple_of(s * ZROW, 8)

    pltpu.sync_copy(src_hbm.at[s], src_v)
    pltpu.sync_copy(dst_hbm.at[s], dst_v)

    bps = nblk // NCORE

    def _xrow(j, carry):
        for u in range(CHUNK // 16):
            sl = pl.ds(u * 16, 16)
            src_v[j, sl] = src_v[j, sl] * nblk + c * bps
        return carry
    lax.fori_loop(0, KT, _xrow, 0)

    for bb in range(bps):
        blk = c * bps + bb
        c0 = pl.multiple_of(blk * DC, DC)

        if bb > 0:
            def _bump(j, carry):
                for u in range(CHUNK // 16):
                    sl = pl.ds(u * 16, 16)
                    src_v[j, sl] = src_v[j, sl] + 1
                return carry
            lax.fori_loop(0, KT, _bump, 0)

        pltpu.sync_copy(zeros_hbm.at[pl.ds(r0, ZROW), :],
                        acc.at[pl.ds(r0, ZROW), :])
        plsc.subcore_barrier()

        def _chunk(j, carry):
            pltpu.async_copy(x_hbm.at[src_v.at[j]], rows_v, gsem).wait()
            pltpu.sync_copy(rows_v, acc.at[dst_v.at[j]], add=True)
            return carry
        lax.fori_loop(0, KT, _chunk, 0)
        plsc.subcore_barrier()

        pltpu.sync_copy(acc.at[pl.ds(r0, ZROW), :],
                        out_hbm.at[pl.ds(r0, ZROW), pl.ds(c0, DC)])
        plsc.subcore_barrier()

    if with_deg:
        pltpu.sync_copy(zeros_hbm.at[pl.ds(r0, ZROW), :],
                        acc.at[pl.ds(r0, ZROW), :])

        def _fill_ones(i, carry):
            for u in range(DC // 16):
                rows_v[i, pl.ds(u * 16, 16)] = jnp.ones((16,), jnp.float32)
            return carry
        lax.fori_loop(0, CHUNK, _fill_ones, 0)
        plsc.subcore_barrier()

        base = c * HKT

        def _deg_chunk(j, carry):
            pltpu.sync_copy(rows_v, acc.at[dst_v.at[base + j]], add=True)
            return carry
        lax.fori_loop(0, HKT, _deg_chunk, 0)
        plsc.subcore_barrier()

        cdeg = pl.multiple_of(c * DC, DC)
        pltpu.sync_copy(acc.at[pl.ds(r0, ZROW), :],
                        deg_hbm.at[pl.ds(r0, ZROW), pl.ds(cdeg, DC)])
        plsc.subcore_barrier()


def _make_spmm(nblk, with_deg):
    d = nblk * DC
    out_type = [jax.ShapeDtypeStruct((NACC, d), jnp.float32)]
    if with_deg:
        out_type.append(jax.ShapeDtypeStruct((NACC, 2 * DC), jnp.float32))
    scratch = [
        pltpu.VMEM((KT, CHUNK), jnp.int32),
        pltpu.VMEM((KT, CHUNK), jnp.int32),
        pltpu.VMEM((CHUNK, DC), jnp.float32),
        pltpu.VMEM_SHARED((NACC, DC), jnp.float32),
        pltpu.SemaphoreType.DMA,
    ]
    mesh = plsc.VectorSubcoreMesh(core_axis_name="c", subcore_axis_name="s")
    return pl.kernel(
        functools.partial(_spmm_body, nblk, with_deg),
        out_type=tuple(out_type),
        mesh=mesh,
        scratch_types=scratch,
    )


RT = 400


def _deg_col(deg_ref):
    return jnp.maximum(deg_ref[:, 0:1] + deg_ref[:, DC:DC + 1], 1.0)


def _tc0_body(agg_ref, deg_ref, w0_ref, b0_ref, w1_ref, h0_ref, z1_ref):
    x = agg_ref[...] / _deg_col(deg_ref)
    h0 = jnp.dot(x, w0_ref[...], preferred_element_type=jnp.float32)
    h0 = jnp.maximum(h0 + b0_ref[...], 0.0)
    h0_ref[...] = h0
    z1_ref[...] = jnp.dot(h0, w1_ref[...], preferred_element_type=jnp.float32)


def _tc1_body(agg_ref, deg_ref, b1_ref, h0_ref, wo_ref, z2_ref):
    h = jnp.maximum(agg_ref[...] / _deg_col(deg_ref) + b1_ref[...], 0.0)
    h = h + h0_ref[...]
    z2_ref[...] = jnp.dot(h, wo_ref[...], preferred_element_type=jnp.float32)


def _tc2_body(agg_ref, deg_ref, bo_ref, out_ref):
    out_ref[...] = agg_ref[...] / _deg_col(deg_ref) + bo_ref[...]


def _row_spec(cols):
    return pl.BlockSpec((RT, cols), lambda r: (r, 0))


def _full_spec(rows, cols):
    return pl.BlockSpec((rows, cols), lambda r: (0, 0))


def _tc0(agg0, deg, W0, b0, W1):
    return pl.pallas_call(
        _tc0_body,
        grid=(N // RT,),
        in_specs=[_row_spec(256), _row_spec(2 * DC), _full_spec(256, 512),
                  _full_spec(1, 512), _full_spec(512, 512)],
        out_specs=[_row_spec(512), _row_spec(512)],
        out_shape=[jax.ShapeDtypeStruct((N, 512), jnp.float32),
                   jax.ShapeDtypeStruct((N, 512), jnp.float32)],
    )(agg0, deg, W0, b0, W1)


def _tc1(agg1, deg, b1, h0, Wo):
    return pl.pallas_call(
        _tc1_body,
        grid=(N // RT,),
        in_specs=[_row_spec(512), _row_spec(2 * DC), _full_spec(1, 512),
                  _row_spec(512), _full_spec(512, 256)],
        out_specs=_row_spec(256),
        out_shape=jax.ShapeDtypeStruct((N, 256), jnp.float32),
    )(agg1, deg, b1, h0, Wo)


def _tc2(agg2, deg, bo):
    return pl.pallas_call(
        _tc2_body,
        grid=(N // RT,),
        in_specs=[_row_spec(256), _row_spec(2 * DC), _full_spec(1, 256)],
        out_specs=_row_spec(256),
        out_shape=jax.ShapeDtypeStruct((N, 256), jnp.float32),
    )(agg2, deg, bo)


def kernel(feats, edge_index, W0, b0, W1, b1, Wo, bo):
    src = edge_index[0]
    dst = edge_index[1]
    pad = EPAD - E
    pad_dst = N + (jnp.arange(pad, dtype=jnp.int32) % (NACC - 1 - N))
    pad_src = jnp.arange(pad, dtype=jnp.int32) % N
    src2 = jnp.concatenate([src, pad_src]).reshape(NS, KT, CHUNK)
    dst2 = jnp.concatenate([dst, pad_dst]).reshape(NS, KT, CHUNK)
    zeros = jnp.zeros((NACC, DC), jnp.float32)

    agg0, deg = _make_spmm(2, True)(
        feats.reshape(N * 2, DC), src2, dst2, zeros)
    h0, z1 = _tc0(agg0, deg, W0, b0.reshape(1, -1), W1)
    (agg1,) = _make_spmm(4, False)(z1.reshape(N * 4, DC), src2, dst2, zeros)
    z2 = _tc1(agg1, deg, b1.reshape(1, -1), h0, Wo)
    (agg2,) = _make_spmm(2, False)(z2.reshape(N * 2, DC), src2, dst2, zeros)
    return _tc2(agg2, deg, bo.reshape(1, -1))


# speedup vs baseline: 2.5657x; 1.4165x over previous
import functools

import jax
import jax.numpy as jnp
from jax import lax
from jax.experimental import pallas as pl
from jax.experimental.pallas import tpu as pltpu
from jax.experimental.pallas import tpu_sc as plsc

N = 10000
E = 160000
DC = 128
NS = 16
NCORE = 2
CHUNK = 128
KT = 80
HKT = KT // 2
EPAD = NS * KT * CHUNK
NACC = 10112
ZROW = NACC // NS


def _spmm_body(nblk, with_deg, *refs):
    if with_deg:
        (x_hbm, src_hbm, dst_hbm, zeros_hbm, out_hbm, deg_hbm,
         src_v, dst_v, r0v, r1v, acc, g0, g1) = refs
    else:
        (x_hbm, src_hbm, dst_hbm, zeros_hbm, out_hbm,
         src_v, dst_v, r0v, r1v, acc, g0, g1) = refs
    rows = (r0v, r1v)
    gsem = (g0, g1)

    c = lax.axis_index("c")
    s = lax.axis_index("s")
    r0 = pl.multiple_of(s * ZROW, 8)

    def _fire_g(chunk_idx, b):
        pltpu.async_copy(x_hbm.at[src_v.at[chunk_idx]], rows[b], gsem[b])

    def _wait_g(b):
        pltpu.make_async_copy(x_hbm.at[src_v.at[0]], rows[b], gsem[b]).wait()

    def _scat(chunk_idx, b):
        pltpu.sync_copy(rows[b], acc.at[dst_v.at[chunk_idx]], add=True)

    bps = nblk // NCORE

    for bb in range(bps):
        blk = c * bps + bb
        c0 = pl.multiple_of(blk * DC, DC)

        pltpu.sync_copy(zeros_hbm.at[pl.ds(r0, ZROW), :],
                        acc.at[pl.ds(r0, ZROW), :])
        plsc.subcore_barrier()

        for half in range(2):
            hof = pl.multiple_of(half * HKT, 8)
            pltpu.sync_copy(src_hbm.at[s, pl.ds(hof, HKT), :], src_v)
            pltpu.sync_copy(dst_hbm.at[s, pl.ds(hof, HKT), :], dst_v)

            def _xrow(j, carry):
                for u in range(CHUNK // 16):
                    sl = pl.ds(u * 16, 16)
                    src_v[j, sl] = src_v[j, sl] * nblk + blk
                return carry
            lax.fori_loop(0, HKT, _xrow, 0)

            for b in range(2):
                _fire_g(b, b)

            def _group(jg, carry):
                cbase = jg * 2
                for b in range(2):
                    _wait_g(b)
                    _scat(cbase + b, b)
                    _fire_g(cbase + 2 + b, b)
                return carry
            lax.fori_loop(0, HKT // 2 - 1, _group, 0)
            for b in range(2):
                _wait_g(b)
                _scat(HKT - 2 + b, b)
        plsc.subcore_barrier()

        pltpu.sync_copy(acc.at[pl.ds(r0, ZROW), :],
                        out_hbm.at[pl.ds(r0, ZROW), pl.ds(c0, DC)])
        plsc.subcore_barrier()

    if with_deg:
        pltpu.sync_copy(zeros_hbm.at[pl.ds(r0, ZROW), :],
                        acc.at[pl.ds(r0, ZROW), :])
        hc = pl.multiple_of(c * HKT, 8)
        pltpu.sync_copy(dst_hbm.at[s, pl.ds(hc, HKT), :], dst_v)

        def _fill_ones(i, carry):
            for u in range(DC // 16):
                r0v[i, pl.ds(u * 16, 16)] = jnp.ones((16,), jnp.float32)
            return carry
        lax.fori_loop(0, CHUNK, _fill_ones, 0)
        plsc.subcore_barrier()

        def _deg_chunk(j, carry):
            pltpu.sync_copy(r0v, acc.at[dst_v.at[j]], add=True)
            return carry
        lax.fori_loop(0, HKT, _deg_chunk, 0)
        plsc.subcore_barrier()

        cdeg = pl.multiple_of(c * DC, DC)
        pltpu.sync_copy(acc.at[pl.ds(r0, ZROW), :],
                        deg_hbm.at[pl.ds(r0, ZROW), pl.ds(cdeg, DC)])
        plsc.subcore_barrier()


def _make_spmm(nblk, with_deg):
    d = nblk * DC
    out_type = [jax.ShapeDtypeStruct((NACC, d), jnp.float32)]
    if with_deg:
        out_type.append(jax.ShapeDtypeStruct((NACC, 2 * DC), jnp.float32))
    scratch = [
        pltpu.VMEM((HKT, CHUNK), jnp.int32),
        pltpu.VMEM((HKT, CHUNK), jnp.int32),
        pltpu.VMEM((CHUNK, DC), jnp.float32),
        pltpu.VMEM((CHUNK, DC), jnp.float32),
        pltpu.VMEM_SHARED((NACC, DC), jnp.float32),
        pltpu.SemaphoreType.DMA,
        pltpu.SemaphoreType.DMA,
    ]
    mesh = plsc.VectorSubcoreMesh(core_axis_name="c", subcore_axis_name="s")
    return pl.kernel(
        functools.partial(_spmm_body, nblk, with_deg),
        out_type=tuple(out_type),
        mesh=mesh,
        scratch_types=scratch,
    )


RT = 400


def _deg_col(deg_ref):
    return jnp.maximum(deg_ref[:, 0:1] + deg_ref[:, DC:DC + 1], 1.0)


def _tc0_body(agg_ref, deg_ref, w0_ref, b0_ref, w1_ref, h0_ref, z1_ref):
    x = agg_ref[...] / _deg_col(deg_ref)
    h0 = jnp.dot(x, w0_ref[...], preferred_element_type=jnp.float32)
    h0 = jnp.maximum(h0 + b0_ref[...], 0.0)
    h0_ref[...] = h0
    z1_ref[...] = jnp.dot(h0, w1_ref[...], preferred_element_type=jnp.float32)


def _tc1_body(agg_ref, deg_ref, b1_ref, h0_ref, wo_ref, z2_ref):
    h = jnp.maximum(agg_ref[...] / _deg_col(deg_ref) + b1_ref[...], 0.0)
    h = h + h0_ref[...]
    z2_ref[...] = jnp.dot(h, wo_ref[...], preferred_element_type=jnp.float32)


def _tc2_body(agg_ref, deg_ref, bo_ref, out_ref):
    out_ref[...] = agg_ref[...] / _deg_col(deg_ref) + bo_ref[...]


def _row_spec(cols):
    return pl.BlockSpec((RT, cols), lambda r: (r, 0))


def _full_spec(rows, cols):
    return pl.BlockSpec((rows, cols), lambda r: (0, 0))


def _tc0(agg0, deg, W0, b0, W1):
    return pl.pallas_call(
        _tc0_body,
        grid=(N // RT,),
        in_specs=[_row_spec(256), _row_spec(2 * DC), _full_spec(256, 512),
                  _full_spec(1, 512), _full_spec(512, 512)],
        out_specs=[_row_spec(512), _row_spec(512)],
        out_shape=[jax.ShapeDtypeStruct((N, 512), jnp.float32),
                   jax.ShapeDtypeStruct((N, 512), jnp.float32)],
    )(agg0, deg, W0, b0, W1)


def _tc1(agg1, deg, b1, h0, Wo):
    return pl.pallas_call(
        _tc1_body,
        grid=(N // RT,),
        in_specs=[_row_spec(512), _row_spec(2 * DC), _full_spec(1, 512),
                  _row_spec(512), _full_spec(512, 256)],
        out_specs=_row_spec(256),
        out_shape=jax.ShapeDtypeStruct((N, 256), jnp.float32),
    )(agg1, deg, b1, h0, Wo)


def _tc2(agg2, deg, bo):
    return pl.pallas_call(
        _tc2_body,
        grid=(N // RT,),
        in_specs=[_row_spec(256), _row_spec(2 * DC), _full_spec(1, 256)],
        out_specs=_row_spec(256),
        out_shape=jax.ShapeDtypeStruct((N, 256), jnp.float32),
    )(agg2, deg, bo)


def kernel(feats, edge_index, W0, b0, W1, b1, Wo, bo):
    src = edge_index[0]
    dst = edge_index[1]
    pad = EPAD - E
    pad_dst = N + (jnp.arange(pad, dtype=jnp.int32) % (NACC - 1 - N))
    pad_src = jnp.arange(pad, dtype=jnp.int32) % N
    src2 = jnp.concatenate([src, pad_src]).reshape(NS, KT, CHUNK)
    dst2 = jnp.concatenate([dst, pad_dst]).reshape(NS, KT, CHUNK)
    zeros = jnp.zeros((NACC, DC), jnp.float32)

    agg0, deg = _make_spmm(2, True)(
        feats.reshape(N * 2, DC), src2, dst2, zeros)
    h0, z1 = _tc0(agg0, deg, W0, b0.reshape(1, -1), W1)
    (agg1,) = _make_spmm(4, False)(z1.reshape(N * 4, DC), src2, dst2, zeros)
    z2 = _tc1(agg1, deg, b1.reshape(1, -1), h0, Wo)
    (agg2,) = _make_spmm(2, False)(z2.reshape(N * 2, DC), src2, dst2, zeros)
    return _tc2(agg2, deg, bo.reshape(1, -1))


# speedup vs baseline: 2.7020x; 1.0531x over previous
import functools

import jax
import jax.numpy as jnp
from jax import lax
from jax.experimental import pallas as pl
from jax.experimental.pallas import tpu as pltpu
from jax.experimental.pallas import tpu_sc as plsc

N = 10000
E = 160000
DC = 128
NS = 16
NCORE = 2
CHUNK = 128
KT = 80
HKT = KT // 2
EPAD = NS * KT * CHUNK
NACC = 10112
ZROW = NACC // NS


def _spmm_body(nblk, with_deg, *refs):
    if with_deg:
        (x_hbm, src_hbm, dst_hbm, zeros_hbm, out_hbm, deg_hbm,
         src_v, dst_v, r0v, r1v, acc, g0, g1) = refs
    else:
        (x_hbm, src_hbm, dst_hbm, zeros_hbm, out_hbm,
         src_v, dst_v, r0v, r1v, acc, g0, g1) = refs
    rows = (r0v, r1v)
    gsem = (g0, g1)

    c = lax.axis_index("c")
    s = lax.axis_index("s")
    r0 = pl.multiple_of(s * ZROW, 8)

    def _fire_g(chunk_idx, b):
        pltpu.async_copy(x_hbm.at[src_v.at[chunk_idx]], rows[b], gsem[b])

    def _wait_g(b):
        pltpu.make_async_copy(x_hbm.at[src_v.at[0]], rows[b], gsem[b]).wait()

    def _scat(chunk_idx, b):
        pltpu.sync_copy(rows[b], acc.at[dst_v.at[chunk_idx]], add=True)

    bps = nblk // NCORE

    for bb in range(bps):
        blk = c * bps + bb
        c0 = pl.multiple_of(blk * DC, DC)

        pltpu.sync_copy(zeros_hbm.at[pl.ds(r0, ZROW), :],
                        acc.at[pl.ds(r0, ZROW), :])
        plsc.subcore_barrier()

        for half in range(2):
            hof = pl.multiple_of(half * HKT, 8)
            pltpu.sync_copy(src_hbm.at[s, pl.ds(hof, HKT), :], src_v)
            pltpu.sync_copy(dst_hbm.at[s, pl.ds(hof, HKT), :], dst_v)

            def _xrow(j, carry):
                for u in range(CHUNK // 16):
                    sl = pl.ds(u * 16, 16)
                    src_v[j, sl] = src_v[j, sl] * nblk + blk
                return carry
            lax.fori_loop(0, HKT, _xrow, 0)

            for b in range(2):
                _fire_g(b, b)

            def _group(jg, carry):
                cbase = jg * 2
                for b in range(2):
                    _wait_g(b)
                    _scat(cbase + b, b)
                    _fire_g(cbase + 2 + b, b)
                return carry
            lax.fori_loop(0, HKT // 2 - 1, _group, 0)
            for b in range(2):
                _wait_g(b)
                _scat(HKT - 2 + b, b)
        plsc.subcore_barrier()

        pltpu.sync_copy(acc.at[pl.ds(r0, ZROW), :],
                        out_hbm.at[pl.ds(r0, ZROW), pl.ds(c0, DC)])
        plsc.subcore_barrier()

    if with_deg:
        pltpu.sync_copy(zeros_hbm.at[pl.ds(r0, ZROW), :],
                        acc.at[pl.ds(r0, ZROW), :])
        hc = pl.multiple_of(c * HKT, 8)
        pltpu.sync_copy(dst_hbm.at[s, pl.ds(hc, HKT), :], dst_v)

        def _fill_ones(i, carry):
            for u in range(DC // 16):
                r0v[i, pl.ds(u * 16, 16)] = jnp.ones((16,), jnp.float32)
            return carry
        lax.fori_loop(0, CHUNK, _fill_ones, 0)
        plsc.subcore_barrier()

        def _deg_chunk(j, carry):
            pltpu.sync_copy(r0v, acc.at[dst_v.at[j]], add=True)
            return carry
        lax.fori_loop(0, HKT, _deg_chunk, 0)
        plsc.subcore_barrier()

        cdeg = pl.multiple_of(c * DC, DC)
        pltpu.sync_copy(acc.at[pl.ds(r0, ZROW), :],
                        deg_hbm.at[pl.ds(r0, ZROW), pl.ds(cdeg, DC)])
        plsc.subcore_barrier()


def _make_spmm(nblk, with_deg):
    d = nblk * DC
    out_type = [jax.ShapeDtypeStruct((NACC, d), jnp.float32)]
    if with_deg:
        out_type.append(jax.ShapeDtypeStruct((NACC, 2 * DC), jnp.float32))
    scratch = [
        pltpu.VMEM((HKT, CHUNK), jnp.int32),
        pltpu.VMEM((HKT, CHUNK), jnp.int32),
        pltpu.VMEM((CHUNK, DC), jnp.float32),
        pltpu.VMEM((CHUNK, DC), jnp.float32),
        pltpu.VMEM_SHARED((NACC, DC), jnp.float32),
        pltpu.SemaphoreType.DMA,
        pltpu.SemaphoreType.DMA,
    ]
    mesh = plsc.VectorSubcoreMesh(core_axis_name="c", subcore_axis_name="s")
    return pl.kernel(
        functools.partial(_spmm_body, nblk, with_deg),
        out_type=tuple(out_type),
        mesh=mesh,
        scratch_types=scratch,
    )


RT = 2000


def _deg_col(deg_ref):
    return jnp.maximum(deg_ref[:, 0:1] + deg_ref[:, DC:DC + 1], 1.0)


def _tc0_body(agg_ref, deg_ref, w0_ref, b0_ref, w1_ref, h0_ref, z1_ref):
    x = agg_ref[...] / _deg_col(deg_ref)
    h0 = jnp.dot(x, w0_ref[...], preferred_element_type=jnp.float32)
    h0 = jnp.maximum(h0 + b0_ref[...], 0.0)
    h0_ref[...] = h0
    z1_ref[...] = jnp.dot(h0, w1_ref[...], preferred_element_type=jnp.float32)


def _tc1_body(agg_ref, deg_ref, b1_ref, h0_ref, wo_ref, z2_ref):
    h = jnp.maximum(agg_ref[...] / _deg_col(deg_ref) + b1_ref[...], 0.0)
    h = h + h0_ref[...]
    z2_ref[...] = jnp.dot(h, wo_ref[...], preferred_element_type=jnp.float32)


def _tc2_body(agg_ref, deg_ref, bo_ref, out_ref):
    out_ref[...] = agg_ref[...] / _deg_col(deg_ref) + bo_ref[...]


def _row_spec(cols):
    return pl.BlockSpec((RT, cols), lambda r: (r, 0))


def _full_spec(rows, cols):
    return pl.BlockSpec((rows, cols), lambda r: (0, 0))


def _tc0(agg0, deg, W0, b0, W1):
    return pl.pallas_call(
        _tc0_body,
        grid=(N // RT,),
        in_specs=[_row_spec(256), _row_spec(2 * DC), _full_spec(256, 512),
                  _full_spec(1, 512), _full_spec(512, 512)],
        out_specs=[_row_spec(512), _row_spec(512)],
        out_shape=[jax.ShapeDtypeStruct((N, 512), jnp.float32),
                   jax.ShapeDtypeStruct((N, 512), jnp.float32)],
    )(agg0, deg, W0, b0, W1)


def _tc1(agg1, deg, b1, h0, Wo):
    return pl.pallas_call(
        _tc1_body,
        grid=(N // RT,),
        in_specs=[_row_spec(512), _row_spec(2 * DC), _full_spec(1, 512),
                  _row_spec(512), _full_spec(512, 256)],
        out_specs=_row_spec(256),
        out_shape=jax.ShapeDtypeStruct((N, 256), jnp.float32),
    )(agg1, deg, b1, h0, Wo)


def _tc2(agg2, deg, bo):
    return pl.pallas_call(
        _tc2_body,
        grid=(N // RT,),
        in_specs=[_row_spec(256), _row_spec(2 * DC), _full_spec(1, 256)],
        out_specs=_row_spec(256),
        out_shape=jax.ShapeDtypeStruct((N, 256), jnp.float32),
    )(agg2, deg, bo)


def kernel(feats, edge_index, W0, b0, W1, b1, Wo, bo):
    src = edge_index[0]
    dst = edge_index[1]
    pad = EPAD - E
    pad_dst = N + (jnp.arange(pad, dtype=jnp.int32) % (NACC - 1 - N))
    pad_src = jnp.arange(pad, dtype=jnp.int32) % N
    src2 = jnp.concatenate([src, pad_src]).reshape(NS, KT, CHUNK)
    dst2 = jnp.concatenate([dst, pad_dst]).reshape(NS, KT, CHUNK)
    zeros = jnp.zeros((NACC, DC), jnp.float32)

    agg0, deg = _make_spmm(2, True)(
        feats.reshape(N * 2, DC), src2, dst2, zeros)
    h0, z1 = _tc0(agg0, deg, W0, b0.reshape(1, -1), W1)
    (agg1,) = _make_spmm(4, False)(z1.reshape(N * 4, DC), src2, dst2, zeros)
    z2 = _tc1(agg1, deg, b1.reshape(1, -1), h0, Wo)
    (agg2,) = _make_spmm(2, False)(z2.reshape(N * 2, DC), src2, dst2, zeros)
    return _tc2(agg2, deg, bo.reshape(1, -1))
